# Initial kernel scaffold; baseline (speedup 1.0000x reference)
import functools

import jax
import jax.numpy as jnp
from jax import lax
from jax.experimental import pallas as pl
from jax.experimental.pallas import tpu as pltpu
from jax.experimental.pallas import tpu_sc as plsc

N = 10000
C = 256
KV = 27
NC = 2
NS = 16
NW = NC * NS
L = 16

CHK = 128
CH1 = 66
CH2 = 132
MPAD = NW * CH1 * CHK
NPAD = 10240
SLAB = NPAD // NS
CHALF = C // 2

_mesh = plsc.VectorSubcoreMesh(core_axis_name="c", subcore_axis_name="s")


def _derive_idx(kq0_v, kq1_v, key_v, kidx_v, out_v):
    def body(g, _):
        a = kq0_v[pl.ds(g * L, L)]
        key_v[pl.ds(g * L, L)] = lax.div(a, KV)
        if kidx_v is not None:
            kidx_v[pl.ds(g * L, L)] = lax.rem(a, KV)
        if out_v is not None:
            out_v[pl.ds(g * L, L)] = kq1_v[pl.ds(g * L, L)]
        return 0

    lax.fori_loop(0, CHK // L, body, 0)


@functools.partial(
    pl.kernel,
    out_type=[
        jax.ShapeDtypeStruct((MPAD,), jnp.float32),
        jax.ShapeDtypeStruct((NW, NPAD), jnp.float32),
    ],
    mesh=_mesh,
    scratch_types=[
        pltpu.VMEM((KV, C), jnp.float32),
        pltpu.VMEM((CHK,), jnp.int32),
        pltpu.VMEM((CHK,), jnp.int32),
        pltpu.VMEM((CHK,), jnp.int32),
        pltpu.VMEM((CHK,), jnp.int32),
        pltpu.VMEM((CHK,), jnp.int32),
        pltpu.VMEM((CHK, C), jnp.float32),
        pltpu.VMEM((CHK, C), jnp.float32),
        pltpu.VMEM((CHK,), jnp.float32),
        pltpu.VMEM((NPAD,), jnp.float32),
        pltpu.SemaphoreType.DMA,
        pltpu.SemaphoreType.DMA,
    ],
)
def _k1(q_hbm, k_hbm, pos_hbm, kq0_hbm, kq1_hbm, e_hbm, dpart_hbm,
        pos_v, kq0_v, kq1_v, key_v, kidx_v, out_v, qrows_v, krows_v, le_v,
        denom_v, sem_q, sem_k):
    cid = lax.axis_index("c")
    sid = lax.axis_index("s")
    wid = sid * NC + cid

    pltpu.sync_copy(pos_hbm, pos_v)

    z16 = jnp.zeros((L,), jnp.float32)

    def zero_body(g, _):
        denom_v[pl.ds(g * L, L)] = z16
        return 0

    lax.fori_loop(0, NPAD // L, zero_body, 0)

    base = wid * (CH1 * CHK)

    def chunk_body(ci, _):
        off = base + ci * CHK
        pltpu.sync_copy(kq0_hbm.at[pl.ds(off, CHK)], kq0_v)
        pltpu.sync_copy(kq1_hbm.at[pl.ds(off, CHK)], kq1_v)
        _derive_idx(kq0_v, kq1_v, key_v, kidx_v, out_v)

        cp_q = pltpu.async_copy(q_hbm.at[out_v], qrows_v, sem_q)
        cp_k = pltpu.async_copy(k_hbm.at[key_v], krows_v, sem_k)
        cp_q.wait()
        cp_k.wait()

        def pair_body(p, _):
            kidx = kidx_v[p]
            acc = jnp.zeros((L,), jnp.float32)
            for j in range(C // L):
                sl = pl.ds(j * L, L)
                acc = acc + qrows_v[p, sl] * (krows_v[p, sl] + pos_v[kidx, sl])
            le_v[p] = jnp.sum(acc, axis=0)
            return 0

        lax.fori_loop(0, CHK, pair_body, 0)

        def exp_body(g, _):
            sl = pl.ds(g * L, L)
            le_v[sl] = jnp.exp(le_v[sl] * 0.0625)
            return 0

        lax.fori_loop(0, CHK // L, exp_body, 0)
        pltpu.sync_copy(le_v, e_hbm.at[pl.ds(off, CHK)])

        def denom_body(p, _):
            o = out_v[p]
            denom_v[o] = denom_v[o] + le_v[p]
            return 0

        lax.fori_loop(0, CHK, denom_body, 0)
        return 0

    lax.fori_loop(0, CH1, chunk_body, 0)
    pltpu.sync_copy(denom_v, dpart_hbm.at[wid])


@functools.partial(
    pl.kernel,
    out_type=jax.ShapeDtypeStruct((NC, NPAD, CHALF), jnp.float32),
    mesh=_mesh,
    scratch_types=[
        pltpu.VMEM((CHK,), jnp.int32),
        pltpu.VMEM((CHK,), jnp.int32),
        pltpu.VMEM((CHK,), jnp.int32),
        pltpu.VMEM((CHK,), jnp.float32),
        pltpu.VMEM((CHK, CHALF), jnp.float32),
        pltpu.VMEM((NPAD,), jnp.float32),
        pltpu.VMEM((4, NPAD), jnp.float32),
        pltpu.VMEM_SHARED((NPAD, CHALF), jnp.float32),
        pltpu.SemaphoreType.DMA,
    ],
)
def _k2(kq0_hbm, kq1_hbm, e_hbm, dpart_hbm, vh_hbm, outp_hbm,
        kq0_v, key_v, out_v, e_v, vrows_v, denom_v, dtmp_v, out_sh, sem):
    cid = lax.axis_index("c")
    sid = lax.axis_index("s")

    z16 = jnp.zeros((L,), jnp.float32)

    def dzero_body(g, _):
        denom_v[pl.ds(g * L, L)] = z16
        return 0

    lax.fori_loop(0, NPAD // L, dzero_body, 0)

    def dsum_outer(t, _):
        pltpu.sync_copy(dpart_hbm.at[pl.ds(t * 4, 4)], dtmp_v)

        def dsum_inner(g, _):
            sl = pl.ds(g * L, L)
            acc = denom_v[sl]
            for r in range(4):
                acc = acc + dtmp_v[r, sl]
            denom_v[sl] = acc
            return 0

        lax.fori_loop(0, NPAD // L, dsum_inner, 0)
        return 0

    lax.fori_loop(0, NW // 4, dsum_outer, 0)

    def vzero_body(r, _):
        for j in range(CHALF // L):
            vrows_v[r, pl.ds(j * L, L)] = z16
        return 0

    lax.fori_loop(0, CHK, vzero_body, 0)
    for jj in range(SLAB // CHK):
        pltpu.sync_copy(vrows_v, out_sh.at[pl.ds(sid * SLAB + jj * CHK, CHK)])
    plsc.subcore_barrier()

    base = sid * (CH2 * CHK)

    def chunk_body(ci, _):
        off = base + ci * CHK
        pltpu.sync_copy(kq0_hbm.at[pl.ds(off, CHK)], kq0_v)
        pltpu.sync_copy(kq1_hbm.at[pl.ds(off, CHK)], out_v)
        _derive_idx(kq0_v, None, key_v, None, None)
        pltpu.sync_copy(e_hbm.at[pl.ds(off, CHK)], e_v)

        def attn_body(g, _):
            sl = pl.ds(g * L, L)
            o16 = out_v[sl]
            d16 = plsc.load_gather(denom_v, [o16])
            e_v[sl] = e_v[sl] / d16
            return 0

        lax.fori_loop(0, CHK // L, attn_body, 0)

        cp = pltpu.async_copy(vh_hbm.at[cid].at[key_v], vrows_v, sem)
        cp.wait()

        def scale_body(p, _):
            a = jnp.full((L,), e_v[p], jnp.float32)
            for j in range(CHALF // L):
                sl = pl.ds(j * L, L)
                vrows_v[p, sl] = vrows_v[p, sl] * a
            return 0

        lax.fori_loop(0, CHK, scale_body, 0)
        pltpu.sync_copy(vrows_v, out_sh.at[out_v], add=True)
        return 0

    lax.fori_loop(0, CH2, chunk_body, 0)
    plsc.subcore_barrier()

    for jj in range(SLAB // CHK):
        r0 = sid * SLAB + jj * CHK
        pltpu.sync_copy(out_sh.at[pl.ds(r0, CHK)], outp_hbm.at[cid, pl.ds(r0, CHK)])


def kernel(q, k, v, pos_enc, kq_map):
    kq0 = kq_map[0].astype(jnp.int32)
    kq1 = kq_map[1].astype(jnp.int32)
    m = kq0.shape[0]
    pad = MPAD - m
    kq0p = jnp.concatenate([kq0, jnp.zeros((pad,), jnp.int32)])
    kq1p = jnp.concatenate([kq1, jnp.full((pad,), N, jnp.int32)])
    vh = jnp.stack([v[:, :CHALF], v[:, CHALF:]])
    e, dpart = _k1(q, k, pos_enc, kq0p, kq1p)
    outp = _k2(kq0p, kq1p, e, dpart, vh)
    return jnp.concatenate([outp[0, :N, :], outp[1, :N, :]], axis=1)


# speedup vs baseline: 4.7113x; 4.7113x over previous
import functools

import jax
import jax.numpy as jnp
from jax import lax
from jax.experimental import pallas as pl
from jax.experimental.pallas import tpu as pltpu
from jax.experimental.pallas import tpu_sc as plsc

N = 10000
C = 256
KV = 27
NC = 2
NS = 16
NW = NC * NS
L = 16

AW = 10112
CHK = 128
CH1 = 66
CH2 = 132
MPAD = NW * CH1 * CHK
NPAD = 10240
SLAB = NPAD // NS
CHALF = C // 2
BM = 1000

_mesh = plsc.VectorSubcoreMesh(core_axis_name="c", subcore_axis_name="s")


def _mm_body(q_ref, kt_ref, o_ref):
    o_ref[...] = jnp.dot(q_ref[...], kt_ref[...],
                         preferred_element_type=jnp.float32)


def _mm(q, kcat_t):
    return pl.pallas_call(
        _mm_body,
        grid=(N // BM, AW // 128),
        in_specs=[
            pl.BlockSpec((BM, C), lambda i, j: (i, 0)),
            pl.BlockSpec((C, 128), lambda i, j: (0, j)),
        ],
        out_specs=pl.BlockSpec((BM, 128), lambda i, j: (i, j)),
        out_shape=jax.ShapeDtypeStruct((N, AW), jnp.float32),
    )(q, kcat_t)


@functools.partial(
    pl.kernel,
    out_type=[
        jax.ShapeDtypeStruct((MPAD,), jnp.float32),
        jax.ShapeDtypeStruct((NC * NPAD,), jnp.float32),
    ],
    mesh=_mesh,
    scratch_types=[
        pltpu.VMEM((CHK,), jnp.int32),
        pltpu.VMEM((CHK,), jnp.int32),
        pltpu.VMEM((CHK,), jnp.int32),
        pltpu.VMEM((CHK,), jnp.int32),
        pltpu.VMEM((CHK,), jnp.float32),
        pltpu.VMEM((CHK,), jnp.float32),
        pltpu.VMEM((SLAB,), jnp.float32),
        pltpu.VMEM_SHARED((NPAD,), jnp.float32),
        pltpu.SemaphoreType.DMA,
        pltpu.SemaphoreType.DMA,
    ],
)
def _k1(aw_hbm, kq0_hbm, kq1_hbm, e_hbm, dpart_hbm,
        kq0_v, out_v, f1_v, f2_v, g1_v, g2_v, zslab_v, den_sh, sem1, sem2):
    cid = lax.axis_index("c")
    sid = lax.axis_index("s")
    wid = sid * NC + cid

    z16 = jnp.zeros((L,), jnp.float32)

    def zslab_body(g, _):
        zslab_v[pl.ds(g * L, L)] = z16
        return 0

    lax.fori_loop(0, SLAB // L, zslab_body, 0)
    pltpu.sync_copy(zslab_v, den_sh.at[pl.ds(sid * SLAB, SLAB)])
    plsc.subcore_barrier()

    base = wid * (CH1 * CHK)

    def chunk_body(ci, _):
        off = base + ci * CHK
        pltpu.sync_copy(kq0_hbm.at[pl.ds(off, CHK)], kq0_v)
        pltpu.sync_copy(kq1_hbm.at[pl.ds(off, CHK)], out_v)

        def idx_body(g, _):
            sl = pl.ds(g * L, L)
            a = kq0_v[sl]
            o = out_v[sl]
            ob = jnp.minimum(o, N - 1) * AW
            f1_v[sl] = ob + lax.div(a, KV)
            f2_v[sl] = ob + (N + lax.rem(a, KV))
            return 0

        lax.fori_loop(0, CHK // L, idx_body, 0)

        cp1 = pltpu.async_copy(aw_hbm.at[f1_v], g1_v, sem1)
        cp2 = pltpu.async_copy(aw_hbm.at[f2_v], g2_v, sem2)
        cp1.wait()
        cp2.wait()

        def exp_body(g, _):
            sl = pl.ds(g * L, L)
            g2_v[sl] = jnp.exp((g1_v[sl] + g2_v[sl]) * 0.0625)
            return 0

        lax.fori_loop(0, CHK // L, exp_body, 0)

        pltpu.sync_copy(g2_v, e_hbm.at[pl.ds(off, CHK)])
        pltpu.sync_copy(g2_v, den_sh.at[out_v], add=True)
        return 0

    lax.fori_loop(0, CH1, chunk_body, 0)

    plsc.subcore_barrier()
    r0 = sid * SLAB
    pltpu.sync_copy(den_sh.at[pl.ds(r0, SLAB)],
                    dpart_hbm.at[pl.ds(cid * NPAD + r0, SLAB)])


@functools.partial(
    pl.kernel,
    out_type=jax.ShapeDtypeStruct((NC, NPAD, CHALF), jnp.float32),
    mesh=_mesh,
    scratch_types=[
        pltpu.VMEM((CHK,), jnp.int32),
        pltpu.VMEM((CHK,), jnp.int32),
        pltpu.VMEM((CHK,), jnp.int32),
        pltpu.VMEM((CHK,), jnp.int32),
        pltpu.VMEM((CHK,), jnp.float32),
        pltpu.VMEM((CHK,), jnp.float32),
        pltpu.VMEM((CHK,), jnp.float32),
        pltpu.VMEM((CHK, CHALF), jnp.float32),
        pltpu.VMEM_SHARED((NPAD, CHALF), jnp.float32),
        pltpu.SemaphoreType.DMA,
        pltpu.SemaphoreType.DMA,
        pltpu.SemaphoreType.DMA,
    ],
)
def _k2(kq0_hbm, kq1_hbm, e_hbm, dpart_hbm, vh_hbm, outp_hbm,
        kq0_v, key_v, out_v, out2_v, e_v, d0_v, d1_v, vrows_v, out_sh,
        sem_v, sem_d0, sem_d1):
    cid = lax.axis_index("c")
    sid = lax.axis_index("s")

    z16 = jnp.zeros((L,), jnp.float32)

    def vzero_body(r, _):
        for j in range(CHALF // L):
            vrows_v[r, pl.ds(j * L, L)] = z16
        return 0

    lax.fori_loop(0, CHK, vzero_body, 0)
    for jj in range(SLAB // CHK):
        pltpu.sync_copy(vrows_v, out_sh.at[pl.ds(sid * SLAB + jj * CHK, CHK)])
    plsc.subcore_barrier()

    base = sid * (CH2 * CHK)

    def chunk_body(ci, _):
        off = base + ci * CHK
        pltpu.sync_copy(kq0_hbm.at[pl.ds(off, CHK)], kq0_v)
        pltpu.sync_copy(kq1_hbm.at[pl.ds(off, CHK)], out_v)
        pltpu.sync_copy(e_hbm.at[pl.ds(off, CHK)], e_v)

        def key_body(g, _):
            sl = pl.ds(g * L, L)
            key_v[sl] = lax.div(kq0_v[sl], KV)
            out2_v[sl] = out_v[sl] + NPAD
            return 0

        lax.fori_loop(0, CHK // L, key_body, 0)

        cpd0 = pltpu.async_copy(dpart_hbm.at[out_v], d0_v, sem_d0)
        cpd1 = pltpu.async_copy(dpart_hbm.at[out2_v], d1_v, sem_d1)
        cpv = pltpu.async_copy(vh_hbm.at[cid].at[key_v], vrows_v, sem_v)
        cpd0.wait()
        cpd1.wait()

        def attn_body(g, _):
            sl = pl.ds(g * L, L)
            e_v[sl] = e_v[sl] / (d0_v[sl] + d1_v[sl])
            return 0

        lax.fori_loop(0, CHK // L, attn_body, 0)
        cpv.wait()

        def scale_body(g, _):
            a16 = e_v[pl.ds(g * L, L)]
            for l in range(L):
                row = g * L + l
                av = jnp.full((L,), a16[l], jnp.float32)
                for j in range(CHALF // L):
                    sl = pl.ds(j * L, L)
                    vrows_v[row, sl] = vrows_v[row, sl] * av
            return 0

        lax.fori_loop(0, CHK // L, scale_body, 0)
        pltpu.sync_copy(vrows_v, out_sh.at[out_v], add=True)
        return 0

    lax.fori_loop(0, CH2, chunk_body, 0)
    plsc.subcore_barrier()

    for jj in range(SLAB // CHK):
        r0 = sid * SLAB + jj * CHK
        pltpu.sync_copy(out_sh.at[pl.ds(r0, CHK)], outp_hbm.at[cid, pl.ds(r0, CHK)])


def kernel(q, k, v, pos_enc, kq_map):
    kq0 = kq_map[0].astype(jnp.int32)
    kq1 = kq_map[1].astype(jnp.int32)
    m = kq0.shape[0]
    pad = MPAD - m
    kq0p = jnp.concatenate([kq0, jnp.zeros((pad,), jnp.int32)])
    kq1p = jnp.concatenate([kq1, jnp.full((pad,), N, jnp.int32)])
    kcat_t = jnp.zeros((C, AW), jnp.float32)
    kcat_t = kcat_t.at[:, :N].set(k.T).at[:, N:N + KV].set(pos_enc.T)
    aw = _mm(q, kcat_t).reshape(N * AW)
    vh = jnp.stack([v[:, :CHALF], v[:, CHALF:]])
    e, dpart = _k1(aw, kq0p, kq1p)
    outp = _k2(kq0p, kq1p, e, dpart, vh)
    return jnp.concatenate([outp[0, :N, :], outp[1, :N, :]], axis=1)
